# baseline (speedup 1.0000x reference)
import functools

import jax
import jax.numpy as jnp
from jax import lax
from jax.experimental import pallas as pl
from jax.experimental.pallas import tpu as pltpu
from jax.experimental.pallas import tpu_sc as plsc

_N_POINTS = 300000
_N_WORKERS = 30
_CHUNK = 400
_CHUNKS_PER_W = _N_POINTS // (_N_WORKERS * _CHUNK)
_VECS = _CHUNK // 16

_VOX = (0.05, 0.05, 0.1)
_RMIN = (0.0, -40.0, -3.0)
_GRID = (1408, 1600, 40)


def _bin_one(p, rmin, vs, n):
    q = (p - rmin) / vs
    c = q.astype(jnp.int32)
    v = (q >= 0.0) & (c < n)
    return c, v


def _make_sc_kernel():
    mesh = plsc.VectorSubcoreMesh(core_axis_name="c", subcore_axis_name="s")

    @functools.partial(
        pl.kernel,
        out_type=jax.ShapeDtypeStruct((_N_POINTS, 3), jnp.int32),
        mesh=mesh,
        scratch_types=[
            pltpu.VMEM((_CHUNK, 4), jnp.float32),
            pltpu.VMEM((_CHUNK, 3), jnp.int32),
        ],
        compiler_params=pltpu.CompilerParams(needs_layout_passes=False),
    )
    def voxel_sc(pts_hbm, out_hbm, in_v, out_v):
        wid = lax.axis_index("s") * 2 + lax.axis_index("c")

        @pl.when(wid < _N_WORKERS)
        def _():
            iota = lax.iota(jnp.int32, 16)
            col0 = jnp.zeros((16,), jnp.int32)
            col1 = col0 + 1
            col2 = col0 + 2
            neg1 = jnp.full((16,), -1, jnp.int32)

            def chunk_body(k, carry):
                base = (wid * _CHUNKS_PER_W + k) * _CHUNK
                pltpu.sync_copy(pts_hbm.at[pl.ds(base, _CHUNK)], in_v)

                def body(i, carry2):
                    rows = iota + i * 16
                    x = plsc.load_gather(in_v, [rows, col0])
                    y = plsc.load_gather(in_v, [rows, col1])
                    z = plsc.load_gather(in_v, [rows, col2])
                    cx, vx = _bin_one(x, _RMIN[0], _VOX[0], _GRID[0])
                    cy, vy = _bin_one(y, _RMIN[1], _VOX[1], _GRID[1])
                    cz, vz = _bin_one(z, _RMIN[2], _VOX[2], _GRID[2])
                    valid = vx & vy & vz
                    plsc.store_scatter(out_v, [rows, col0],
                                       jnp.where(valid, cz, neg1))
                    plsc.store_scatter(out_v, [rows, col1],
                                       jnp.where(valid, cy, neg1))
                    plsc.store_scatter(out_v, [rows, col2],
                                       jnp.where(valid, cx, neg1))
                    return carry2

                lax.fori_loop(0, _VECS, body, 0)
                pltpu.sync_copy(out_v, out_hbm.at[pl.ds(base, _CHUNK)])
                return carry

            lax.fori_loop(0, _CHUNKS_PER_W, chunk_body, 0)

    return voxel_sc


_voxel_sc = _make_sc_kernel()


def kernel(input):
    return _voxel_sc(input)


# speedup vs baseline: 1.0112x; 1.0112x over previous
import functools

import jax
import jax.numpy as jnp
from jax import lax
from jax.experimental import pallas as pl
from jax.experimental.pallas import tpu as pltpu
from jax.experimental.pallas import tpu_sc as plsc

_N_POINTS = 300000
_N_WORKERS = 30
_PER_W = _N_POINTS // _N_WORKERS
_CHUNK = 480
_FULL_CHUNKS = _PER_W // _CHUNK
_TAIL = _PER_W - _FULL_CHUNKS * _CHUNK

_VOX = (0.05, 0.05, 0.1)
_RMIN = (0.0, -40.0, -3.0)
_GRID = (1408, 1600, 40)


def _bin_one(p, rmin, vs, n):
    q = (p - rmin) / vs
    c = q.astype(jnp.int32)
    v = (q >= 0.0) & (c < n)
    return c, v


def _make_sc_kernel():
    mesh = plsc.VectorSubcoreMesh(core_axis_name="c", subcore_axis_name="s")

    @functools.partial(
        pl.kernel,
        out_type=jax.ShapeDtypeStruct((_N_POINTS, 3), jnp.int32),
        mesh=mesh,
        scratch_types=[
            pltpu.VMEM((_CHUNK, 4), jnp.float32),
            pltpu.VMEM((_CHUNK, 3), jnp.int32),
        ],
        compiler_params=pltpu.CompilerParams(needs_layout_passes=False),
    )
    def voxel_sc(pts_hbm, out_hbm, in_v, out_v):
        wid = lax.axis_index("s") * 2 + lax.axis_index("c")

        @pl.when(wid < _N_WORKERS)
        def _():
            iota = lax.iota(jnp.int32, 16)
            col0 = jnp.zeros((16,), jnp.int32)
            col1 = col0 + 1
            col2 = col0 + 2
            neg1 = jnp.full((16,), -1, jnp.int32)

            def do_chunk(base, nrows):
                pltpu.sync_copy(pts_hbm.at[pl.ds(base, nrows)],
                                in_v.at[pl.ds(0, nrows)])

                def body(i, carry2):
                    rows = iota + i * 16
                    x = plsc.load_gather(in_v, [rows, col0])
                    y = plsc.load_gather(in_v, [rows, col1])
                    z = plsc.load_gather(in_v, [rows, col2])
                    cx, vx = _bin_one(x, _RMIN[0], _VOX[0], _GRID[0])
                    cy, vy = _bin_one(y, _RMIN[1], _VOX[1], _GRID[1])
                    cz, vz = _bin_one(z, _RMIN[2], _VOX[2], _GRID[2])
                    valid = vx & vy & vz
                    plsc.store_scatter(out_v, [rows, col0],
                                       jnp.where(valid, cz, neg1))
                    plsc.store_scatter(out_v, [rows, col1],
                                       jnp.where(valid, cy, neg1))
                    plsc.store_scatter(out_v, [rows, col2],
                                       jnp.where(valid, cx, neg1))
                    return carry2

                lax.fori_loop(0, nrows // 16, body, 0)
                pltpu.sync_copy(out_v.at[pl.ds(0, nrows)],
                                out_hbm.at[pl.ds(base, nrows)])

            def chunk_body(k, carry):
                do_chunk(wid * _PER_W + k * _CHUNK, _CHUNK)
                return carry

            lax.fori_loop(0, _FULL_CHUNKS, chunk_body, 0)
            do_chunk(wid * _PER_W + _FULL_CHUNKS * _CHUNK, _TAIL)

    return voxel_sc


_voxel_sc = _make_sc_kernel()


def kernel(input):
    return _voxel_sc(input)
